# baseline (speedup 1.0000x reference)
import functools

import jax
import jax.numpy as jnp
from jax import lax
from jax.experimental import pallas as pl
from jax.experimental.pallas import tpu as pltpu
from jax.experimental.pallas import tpu_sc as plsc

_W = 128
_NBUF = 5


def kernel(x, emb):
    B, H = x.shape
    V, D = emb.shape
    info = plsc.get_sparse_core_info()
    nw = info.num_cores * info.num_subcores
    nbt = B // _W
    nd8 = D // 8
    assert nbt == nw and D % 8 == 0 and H % _NBUF == 0

    xt = x.T.reshape(H, nbt, _W).astype(jnp.int32)

    mesh = plsc.VectorSubcoreMesh(core_axis_name="core", subcore_axis_name="subcore")

    @functools.partial(
        pl.kernel,
        out_type=jax.ShapeDtypeStruct((H, nd8, nbt, 8, _W), emb.dtype),
        mesh=mesh,
        scratch_types=[
            pltpu.VMEM((H, _W), jnp.int32),
            pltpu.VMEM((_NBUF, _W, D), jnp.float32),
            pltpu.VMEM((_NBUF, nd8, 8, _W), jnp.float32),
            pltpu.SemaphoreType.DMA((_NBUF,)),
            pltpu.SemaphoreType.DMA((_NBUF,)),
        ],
        compiler_params=pltpu.CompilerParams(
            use_tc_tiling_on_sc=False, needs_layout_passes=False),
    )
    def gather_kernel(emb_hbm, idx_hbm, out_hbm, idx_v, g_v, t_v, gsem, ssem):
        wid = lax.axis_index("subcore") * info.num_cores + lax.axis_index("core")
        pltpu.sync_copy(idx_hbm.at[:, wid], idx_v)

        def gather_start(h, b):
            pltpu.make_async_copy(
                emb_hbm.at[idx_v.at[h, pl.ds(0, 64)]],
                g_v.at[b, pl.ds(0, 64)], gsem.at[b]).start()
            pltpu.make_async_copy(
                emb_hbm.at[idx_v.at[h, pl.ds(64, 64)]],
                g_v.at[b, pl.ds(64, 64)], gsem.at[b]).start()

        def gather_wait(h, b):
            pltpu.make_async_copy(
                emb_hbm.at[idx_v.at[h, pl.ds(0, 64)]],
                g_v.at[b, pl.ds(0, 64)], gsem.at[b]).wait()
            pltpu.make_async_copy(
                emb_hbm.at[idx_v.at[h, pl.ds(64, 64)]],
                g_v.at[b, pl.ds(64, 64)], gsem.at[b]).wait()

        def scatter(h, b):
            return pltpu.make_async_copy(
                t_v.at[b], out_hbm.at[h, :, wid], ssem.at[b])

        lanes = lax.iota(jnp.int32, 16)
        bks = [(lanes + k) & 15 for k in range(16)]
        dvs = [d0 + lanes for d0 in range(0, D, 16)]
        dtvs = [dv >> 3 for dv in dvs]
        d8vs = [dv & 7 for dv in dvs]

        def transpose(b):
            @pl.loop(0, _W, step=32)
            def _(b0):
                for half in range(2):
                    bvs = [b0 + half * 16 + bk for bk in bks]
                    for di in range(D // 16):
                        vecs = [
                            plsc.load_gather(g_v.at[b], [bvs[k], dvs[di]])
                            for k in range(16)
                        ]
                        for k in range(16):
                            plsc.store_scatter(
                                t_v.at[b], [dtvs[di], d8vs[di], bvs[k]], vecs[k])

        for b in range(_NBUF):
            gather_start(b, b)

        for b in range(_NBUF):
            gather_wait(b, b)
            transpose(b)
            scatter(b, b).start()
            gather_start(b + _NBUF, b)

        @pl.loop(_NBUF, H - _NBUF, step=_NBUF)
        def _(h0):
            for b in range(_NBUF):
                h = h0 + b
                gather_wait(h, b)
                scatter(h - _NBUF, b).wait()
                transpose(b)
                scatter(h, b).start()
                gather_start(h + _NBUF, b)

        h0 = H - _NBUF
        for b in range(_NBUF):
            h = h0 + b
            gather_wait(h, b)
            scatter(h - _NBUF, b).wait()
            transpose(b)
            scatter(h, b).start()
        for b in range(_NBUF):
            scatter(h0 + b, b).wait()

    out5 = gather_kernel(emb, xt)
    return out5.transpose(2, 4, 0, 1, 3).reshape(B, H, D)


# speedup vs baseline: 1.0059x; 1.0059x over previous
import functools

import jax
import jax.numpy as jnp
from jax import lax
from jax.experimental import pallas as pl
from jax.experimental.pallas import tpu as pltpu
from jax.experimental.pallas import tpu_sc as plsc

_W = 128
_NBUF = 4


def kernel(x, emb):
    B, H = x.shape
    V, D = emb.shape
    info = plsc.get_sparse_core_info()
    nw = info.num_cores * info.num_subcores
    nbt = B // _W
    nd8 = D // 8
    assert nbt == nw and D % 8 == 0 and H % _NBUF == 0

    xt = x.T.reshape(H, nbt, _W).astype(jnp.int32)

    mesh = plsc.VectorSubcoreMesh(core_axis_name="core", subcore_axis_name="subcore")

    @functools.partial(
        pl.kernel,
        out_type=jax.ShapeDtypeStruct((H, nd8, nbt, 8, _W), emb.dtype),
        mesh=mesh,
        scratch_types=[
            pltpu.VMEM((H, _W), jnp.int32),
            pltpu.VMEM((_NBUF, _W, D), jnp.float32),
            pltpu.VMEM((_NBUF, nd8, 8, _W), jnp.float32),
            pltpu.SemaphoreType.DMA((_NBUF,)),
            pltpu.SemaphoreType.DMA((_NBUF,)),
        ],
        compiler_params=pltpu.CompilerParams(
            use_tc_tiling_on_sc=False, needs_layout_passes=False),
    )
    def gather_kernel(emb_hbm, idx_hbm, out_hbm, idx_v, g_v, t_v, gsem, ssem):
        wid = lax.axis_index("subcore") * info.num_cores + lax.axis_index("core")
        pltpu.sync_copy(idx_hbm.at[:, wid], idx_v)

        def gather_start(h, b):
            pltpu.make_async_copy(
                emb_hbm.at[idx_v.at[h, pl.ds(0, 64)]],
                g_v.at[b, pl.ds(0, 64)], gsem.at[b]).start()
            pltpu.make_async_copy(
                emb_hbm.at[idx_v.at[h, pl.ds(64, 64)]],
                g_v.at[b, pl.ds(64, 64)], gsem.at[b]).start()

        def gather_wait(h, b):
            pltpu.make_async_copy(
                emb_hbm.at[idx_v.at[h, pl.ds(0, 64)]],
                g_v.at[b, pl.ds(0, 64)], gsem.at[b]).wait()
            pltpu.make_async_copy(
                emb_hbm.at[idx_v.at[h, pl.ds(64, 64)]],
                g_v.at[b, pl.ds(64, 64)], gsem.at[b]).wait()

        def scatter(h, b):
            return pltpu.make_async_copy(
                t_v.at[b], out_hbm.at[h, :, wid], ssem.at[b])

        lanes = lax.iota(jnp.int32, 16)
        bks = [(lanes + k) & 15 for k in range(16)]
        dvs = [d0 + lanes for d0 in range(0, D, 16)]
        dtvs = [dv >> 3 for dv in dvs]
        d8vs = [dv & 7 for dv in dvs]

        def transpose(b):
            @pl.loop(0, _W, step=32)
            def _(b0):
                for half in range(2):
                    bvs = [b0 + half * 16 + bk for bk in bks]
                    for di in range(D // 16):
                        vecs = [
                            plsc.load_gather(g_v.at[b], [bvs[k], dvs[di]])
                            for k in range(16)
                        ]
                        for k in range(16):
                            plsc.store_scatter(
                                t_v.at[b], [dtvs[di], d8vs[di], bvs[k]], vecs[k])

        for b in range(_NBUF):
            gather_start(b, b)

        for b in range(_NBUF):
            gather_wait(b, b)
            transpose(b)
            scatter(b, b).start()
            gather_start(b + _NBUF, b)

        @pl.loop(_NBUF, H - _NBUF, step=_NBUF)
        def _(h0):
            for b in range(_NBUF):
                h = h0 + b
                gather_wait(h, b)
                scatter(h - _NBUF, b).wait()
                transpose(b)
                scatter(h, b).start()
                gather_start(h + _NBUF, b)

        h0 = H - _NBUF
        for b in range(_NBUF):
            h = h0 + b
            gather_wait(h, b)
            scatter(h - _NBUF, b).wait()
            transpose(b)
            scatter(h, b).start()
        for b in range(_NBUF):
            scatter(h0 + b, b).wait()

    out5 = gather_kernel(emb, xt)
    return out5.transpose(2, 4, 0, 1, 3).reshape(B, H, D)
